# baseline (speedup 1.0000x reference)
import functools

import jax
import jax.numpy as jnp
from jax import lax
from jax.experimental import pallas as pl
from jax.experimental.pallas import tpu as pltpu
from jax.experimental.pallas import tpu_sc as plsc

B = 8
F = 2048
J = 17
C = 128
P = J * 2
K = 512
NC = 2
NS = 16
BPC = B // NC
QF = F // 4
GQ = QF // 16
GF = F // 16
KPT = K // 4
SCOLS = QF + 8
NBUF = 4


def _body(poses_hbm, tokens_hbm, out_tok, out_idx,
          pose_v, scores_v, sc_all, idx_v, gidx_v, gidx2, hist_v, suf_v,
          bufs, scores_sh, idx_sh, sem, gsems, ssems):
  c = lax.axis_index("c")
  s = lax.axis_index("s")
  lanes = lax.iota(jnp.int32, 16)

  bl = s // 4
  q = s % 4
  b = c * BPC + bl
  off8 = jnp.where(q > 0, 8, 0)
  w0 = q * QF - off8
  pltpu.sync_copy(poses_hbm.at[pl.ds(b * P, P), pl.ds(w0, SCOLS)], pose_v)

  def score_group(g, _):
    colc = off8 + g * 16 + lanes
    colp = jnp.maximum(colc - 1, 0)
    acc0 = jnp.zeros((16,), jnp.float32)
    acc1 = jnp.zeros((16,), jnp.float32)
    for r in range(P):
      rr = jnp.full((16,), r, jnp.int32)
      cur = plsc.load_gather(pose_v, [rr, colc])
      prv = plsc.load_gather(pose_v, [rr, colp])
      if r % 2 == 0:
        acc0 = acc0 + jnp.abs(cur - prv)
      else:
        acc1 = acc1 + jnp.abs(cur - prv)
    scores_v[pl.ds(g * 16, 16)] = acc0 + acc1
    return 0

  lax.fori_loop(0, GQ, score_group, 0)
  pltpu.sync_copy(scores_v,
                  scores_sh.at[pl.ds(pl.multiple_of(bl * F + q * QF, QF), QF)])
  plsc.subcore_barrier()

  @pl.when(s < BPC)
  def _phase_b():
    pltpu.sync_copy(scores_sh.at[pl.ds(pl.multiple_of(s * F, F), F)], sc_all)

    ones16 = jnp.full((16,), 1, jnp.int32)
    zeros16 = jnp.zeros((16,), jnp.int32)
    pref = jnp.int32(0)
    krem = jnp.int32(K)
    rounds = ((None, 23, 0xff), (23, 15, 0xff), (15, 7, 0xff),
              (7, None, 0x7f))
    for keep_sh, dig_sh, dig_mask in rounds:
      def zbody(i, _):
        for u in range(4):
          hist_v[pl.ds((i * 4 + u) * 16, 16)] = zeros16
        return 0
      lax.fori_loop(0, 64, zbody, 0)

      def cbody(i, _):
        for u in range(4):
          g = i * 4 + u
          bits = plsc.bitcast(sc_all[pl.ds(g * 16, 16)], jnp.int32)
          if dig_sh is None:
            digit = bits & dig_mask
          else:
            digit = lax.shift_right_logical(bits, dig_sh) & dig_mask
          mask = (None if keep_sh is None
                  else lax.shift_right_logical(bits, keep_sh) == pref)
          plsc.addupdate_scatter(hist_v, [lanes * 256 + digit], ones16,
                                 mask=mask)
        return 0
      lax.fori_loop(0, GF // 4, cbody, 0)

      cum = jnp.int32(0)
      nsat = jnp.int32(0)
      for t in range(15, -1, -1):
        v = [hist_v[pl.ds(l * 256 + t * 16, 16)] for l in range(16)]
        while len(v) > 1:
          v = [v[i] + v[i + 1] for i in range(0, len(v), 2)]
        tot16 = v[0]
        suf16 = lax.rev(plsc.cumsum(lax.rev(tot16, (0,))), (0,)) + cum
        suf_v[pl.ds(t * 16, 16)] = suf16
        nsat = nsat + jnp.sum(jnp.where(suf16 >= krem, 1, 0))
        cum = jnp.max(suf16)
      dstar = nsat - 1
      nxt = jnp.minimum(dstar + 1, 255)
      s_next = jnp.where(
          dstar >= 255, 0,
          jnp.max(plsc.load_gather(suf_v, [jnp.full((16,), 0, jnp.int32)
                                           + nxt])))
      krem = krem - s_next
      if keep_sh is None:
        pref = dstar
      elif dig_sh is not None:
        pref = (pref << 8) | dstar
      else:
        pref = (pref << 7) | dstar
    thr = pref
    thr_v = jnp.full((16,), thr, jnp.int32)

    def compact(g, carry):
      off, equota = carry
      sv = plsc.bitcast(sc_all[pl.ds(g * 16, 16)], jnp.int32)
      fidx = g * 16 + lanes
      m_gt = sv > thr_v
      m_eq = sv == thr_v
      eq_rank = plsc.cumsum(jnp.where(m_eq, 1, 0))
      m_eq_sel = m_eq & (eq_rank <= equota)
      m = m_gt | m_eq_sel
      mi = jnp.where(m, 1, 0)
      pos = off + plsc.cumsum(mi) - 1
      plsc.store_scatter(idx_v, [pos], fidx, mask=m)
      return (off + jnp.sum(mi),
              equota - jnp.sum(jnp.where(m_eq_sel, 1, 0)))

    lax.fori_loop(0, GF, compact, (jnp.int32(0), krem))
    gb = c * BPC + s
    pltpu.sync_copy(idx_v, out_idx.at[pl.ds(pl.multiple_of(gb * K, K), K)])
    pltpu.sync_copy(idx_v, idx_sh.at[pl.ds(pl.multiple_of(s * K, K), K)])

  plsc.subcore_barrier()

  pltpu.sync_copy(
      idx_sh.at[pl.ds(pl.multiple_of(bl * K + q * KPT, KPT), KPT)], gidx_v)
  for j in range(J):
    rbase = (b * J + j) * F
    for t in range(KPT // 16):
      gidx2[j, pl.ds(t * 16, 16)] = gidx_v[pl.ds(t * 16, 16)] + rbase

  out0 = b * (J * K) + q * KPT
  gathers = []
  stores = []
  for j in range(J):
    gathers.append(pltpu.make_async_copy(
        tokens_hbm.at[gidx2.at[j]], bufs[j % NBUF], gsems[j % NBUF]))
    stores.append(pltpu.make_async_copy(
        bufs[j % NBUF],
        out_tok.at[pl.ds(pl.multiple_of(out0 + j * K, KPT), KPT)],
        ssems[j % NBUF]))
  gathers[0].start()
  for j in range(J):
    if j + 1 < J:
      if j + 1 - NBUF >= 0:
        stores[j + 1 - NBUF].wait()
      gathers[j + 1].start()
    gathers[j].wait()
    stores[j].start()
  for j in range(max(0, J - NBUF), J):
    stores[j].wait()


def _wrapped(poses_hbm, tokens_hbm, out_tok, out_idx,
             pose_v, scores_v, sc_all, idx_v, gidx_v, gidx2, hist_v, suf_v,
             rb0, rb1, rb2, rb3, scores_sh, idx_sh, sem,
             gsem0, gsem1, gsem2, gsem3, ssem0, ssem1, ssem2, ssem3):
  _body(poses_hbm, tokens_hbm, out_tok, out_idx,
        pose_v, scores_v, sc_all, idx_v, gidx_v, gidx2, hist_v, suf_v,
        (rb0, rb1, rb2, rb3), scores_sh, idx_sh, sem,
        (gsem0, gsem1, gsem2, gsem3), (ssem0, ssem1, ssem2, ssem3))


@functools.lru_cache(maxsize=1)
def _build():
  return pl.kernel(
      _wrapped,
      out_type=(jax.ShapeDtypeStruct((B * J * K, C), jnp.float32),
                jax.ShapeDtypeStruct((B * K,), jnp.int32)),
      mesh=plsc.VectorSubcoreMesh(core_axis_name="c", subcore_axis_name="s",
                                  num_cores=NC, num_subcores=NS),
      scratch_types=(
          pltpu.VMEM((P, SCOLS), jnp.float32),
          pltpu.VMEM((QF,), jnp.float32),
          pltpu.VMEM((F,), jnp.float32),
          pltpu.VMEM((K,), jnp.int32),
          pltpu.VMEM((KPT,), jnp.int32),
          pltpu.VMEM((J, KPT), jnp.int32),
          pltpu.VMEM((16 * 256,), jnp.int32),
          pltpu.VMEM((256,), jnp.int32),
          pltpu.VMEM((KPT, C), jnp.float32),
          pltpu.VMEM((KPT, C), jnp.float32),
          pltpu.VMEM((KPT, C), jnp.float32),
          pltpu.VMEM((KPT, C), jnp.float32),
          pltpu.VMEM_SHARED((BPC * F,), jnp.float32),
          pltpu.VMEM_SHARED((BPC * K,), jnp.int32),
          pltpu.SemaphoreType.DMA,
          pltpu.SemaphoreType.DMA,
          pltpu.SemaphoreType.DMA,
          pltpu.SemaphoreType.DMA,
          pltpu.SemaphoreType.DMA,
          pltpu.SemaphoreType.DMA,
          pltpu.SemaphoreType.DMA,
          pltpu.SemaphoreType.DMA,
          pltpu.SemaphoreType.DMA,
      ),
      compiler_params=pltpu.CompilerParams(use_tc_tiling_on_sc=False,
                                           needs_layout_passes=False),
  )


def kernel(tokens, input_2d_poses):
  poses2d = input_2d_poses.transpose(0, 2, 3, 1).reshape(B * P, F)
  tokens_flat = tokens.transpose(0, 2, 1, 3).reshape(B * J * F, C)
  out_tok, out_idx = _build()(poses2d, tokens_flat)
  out = out_tok.reshape(B, J, K, C).transpose(0, 2, 1, 3)
  return out, out_idx.reshape(B, K)


# speedup vs baseline: 1.0033x; 1.0033x over previous
import functools

import jax
import jax.numpy as jnp
from jax import lax
from jax.experimental import pallas as pl
from jax.experimental.pallas import tpu as pltpu
from jax.experimental.pallas import tpu_sc as plsc

B = 8
F = 2048
J = 17
C = 128
P = J * 2
K = 512
NC = 2
NS = 16
BPC = B // NC
QF = F // 4
GQ = QF // 16
GF = F // 16
KPT = K // 4
SCOLS = QF + 8
NBUF = 5


def _body(poses_hbm, tokens_hbm, out_tok, out_idx,
          pose_v, scores_v, sc_all, idx_v, gidx_v, gidx2, hist_v, suf_v,
          bufs, scores_sh, idx_sh, sem, gsems, ssems):
  c = lax.axis_index("c")
  s = lax.axis_index("s")
  lanes = lax.iota(jnp.int32, 16)

  bl = s // 4
  q = s % 4
  b = c * BPC + bl
  off8 = jnp.where(q > 0, 8, 0)
  w0 = q * QF - off8
  pltpu.sync_copy(poses_hbm.at[pl.ds(b * P, P), pl.ds(w0, SCOLS)], pose_v)

  def score_group(g, _):
    colc = off8 + g * 16 + lanes
    colp = jnp.maximum(colc - 1, 0)
    acc0 = jnp.zeros((16,), jnp.float32)
    acc1 = jnp.zeros((16,), jnp.float32)
    for r in range(P):
      rr = jnp.full((16,), r, jnp.int32)
      cur = plsc.load_gather(pose_v, [rr, colc])
      prv = plsc.load_gather(pose_v, [rr, colp])
      if r % 2 == 0:
        acc0 = acc0 + jnp.abs(cur - prv)
      else:
        acc1 = acc1 + jnp.abs(cur - prv)
    scores_v[pl.ds(g * 16, 16)] = acc0 + acc1
    return 0

  lax.fori_loop(0, GQ, score_group, 0)
  pltpu.sync_copy(scores_v,
                  scores_sh.at[pl.ds(pl.multiple_of(bl * F + q * QF, QF), QF)])
  plsc.subcore_barrier()

  @pl.when(s < BPC)
  def _phase_b():
    pltpu.sync_copy(scores_sh.at[pl.ds(pl.multiple_of(s * F, F), F)], sc_all)

    ones16 = jnp.full((16,), 1, jnp.int32)
    zeros16 = jnp.zeros((16,), jnp.int32)
    pref = jnp.int32(0)
    krem = jnp.int32(K)
    rounds = ((None, 23, 0xff), (23, 15, 0xff), (15, 7, 0xff),
              (7, None, 0x7f))
    for keep_sh, dig_sh, dig_mask in rounds:
      def zbody(i, _):
        for u in range(4):
          hist_v[pl.ds((i * 4 + u) * 16, 16)] = zeros16
        return 0
      lax.fori_loop(0, 64, zbody, 0)

      def cbody(i, _):
        for u in range(4):
          g = i * 4 + u
          bits = plsc.bitcast(sc_all[pl.ds(g * 16, 16)], jnp.int32)
          if dig_sh is None:
            digit = bits & dig_mask
          else:
            digit = lax.shift_right_logical(bits, dig_sh) & dig_mask
          mask = (None if keep_sh is None
                  else lax.shift_right_logical(bits, keep_sh) == pref)
          plsc.addupdate_scatter(hist_v, [lanes * 256 + digit], ones16,
                                 mask=mask)
        return 0
      lax.fori_loop(0, GF // 4, cbody, 0)

      cum = jnp.int32(0)
      nsat = jnp.int32(0)
      for t in range(15, -1, -1):
        v = [hist_v[pl.ds(l * 256 + t * 16, 16)] for l in range(16)]
        while len(v) > 1:
          v = [v[i] + v[i + 1] for i in range(0, len(v), 2)]
        tot16 = v[0]
        suf16 = lax.rev(plsc.cumsum(lax.rev(tot16, (0,))), (0,)) + cum
        suf_v[pl.ds(t * 16, 16)] = suf16
        nsat = nsat + jnp.sum(jnp.where(suf16 >= krem, 1, 0))
        cum = jnp.max(suf16)
      dstar = nsat - 1
      nxt = jnp.minimum(dstar + 1, 255)
      s_next = jnp.where(
          dstar >= 255, 0,
          jnp.max(plsc.load_gather(suf_v, [jnp.full((16,), 0, jnp.int32)
                                           + nxt])))
      krem = krem - s_next
      if keep_sh is None:
        pref = dstar
      elif dig_sh is not None:
        pref = (pref << 8) | dstar
      else:
        pref = (pref << 7) | dstar
    thr = pref
    thr_v = jnp.full((16,), thr, jnp.int32)

    def compact(g, carry):
      off, equota = carry
      sv = plsc.bitcast(sc_all[pl.ds(g * 16, 16)], jnp.int32)
      fidx = g * 16 + lanes
      m_gt = sv > thr_v
      m_eq = sv == thr_v
      eq_rank = plsc.cumsum(jnp.where(m_eq, 1, 0))
      m_eq_sel = m_eq & (eq_rank <= equota)
      m = m_gt | m_eq_sel
      mi = jnp.where(m, 1, 0)
      pos = off + plsc.cumsum(mi) - 1
      plsc.store_scatter(idx_v, [pos], fidx, mask=m)
      return (off + jnp.sum(mi),
              equota - jnp.sum(jnp.where(m_eq_sel, 1, 0)))

    lax.fori_loop(0, GF, compact, (jnp.int32(0), krem))
    gb = c * BPC + s
    pltpu.sync_copy(idx_v, out_idx.at[pl.ds(pl.multiple_of(gb * K, K), K)])
    pltpu.sync_copy(idx_v, idx_sh.at[pl.ds(pl.multiple_of(s * K, K), K)])

  plsc.subcore_barrier()

  pltpu.sync_copy(
      idx_sh.at[pl.ds(pl.multiple_of(bl * K + q * KPT, KPT), KPT)], gidx_v)
  for j in range(J):
    rbase = (b * J + j) * F
    for t in range(KPT // 16):
      gidx2[j, pl.ds(t * 16, 16)] = gidx_v[pl.ds(t * 16, 16)] + rbase

  out0 = b * (J * K) + q * KPT
  gathers = []
  stores = []
  for j in range(J):
    gathers.append(pltpu.make_async_copy(
        tokens_hbm.at[gidx2.at[j]], bufs[j % NBUF], gsems[j % NBUF]))
    stores.append(pltpu.make_async_copy(
        bufs[j % NBUF],
        out_tok.at[pl.ds(pl.multiple_of(out0 + j * K, KPT), KPT)],
        ssems[j % NBUF]))
  gathers[0].start()
  for j in range(J):
    if j + 1 < J:
      if j + 1 - NBUF >= 0:
        stores[j + 1 - NBUF].wait()
      gathers[j + 1].start()
    gathers[j].wait()
    stores[j].start()
  for j in range(max(0, J - NBUF), J):
    stores[j].wait()


def _wrapped(poses_hbm, tokens_hbm, out_tok, out_idx,
             pose_v, scores_v, sc_all, idx_v, gidx_v, gidx2, hist_v, suf_v,
             rb0, rb1, rb2, rb3, rb4, scores_sh, idx_sh, sem,
             gsem0, gsem1, gsem2, gsem3, gsem4,
             ssem0, ssem1, ssem2, ssem3, ssem4):
  _body(poses_hbm, tokens_hbm, out_tok, out_idx,
        pose_v, scores_v, sc_all, idx_v, gidx_v, gidx2, hist_v, suf_v,
        (rb0, rb1, rb2, rb3, rb4), scores_sh, idx_sh, sem,
        (gsem0, gsem1, gsem2, gsem3, gsem4),
        (ssem0, ssem1, ssem2, ssem3, ssem4))


@functools.lru_cache(maxsize=1)
def _build():
  return pl.kernel(
      _wrapped,
      out_type=(jax.ShapeDtypeStruct((B * J * K, C), jnp.float32),
                jax.ShapeDtypeStruct((B * K,), jnp.int32)),
      mesh=plsc.VectorSubcoreMesh(core_axis_name="c", subcore_axis_name="s",
                                  num_cores=NC, num_subcores=NS),
      scratch_types=(
          pltpu.VMEM((P, SCOLS), jnp.float32),
          pltpu.VMEM((QF,), jnp.float32),
          pltpu.VMEM((F,), jnp.float32),
          pltpu.VMEM((K,), jnp.int32),
          pltpu.VMEM((KPT,), jnp.int32),
          pltpu.VMEM((J, KPT), jnp.int32),
          pltpu.VMEM((16 * 256,), jnp.int32),
          pltpu.VMEM((256,), jnp.int32),
          pltpu.VMEM((KPT, C), jnp.float32),
          pltpu.VMEM((KPT, C), jnp.float32),
          pltpu.VMEM((KPT, C), jnp.float32),
          pltpu.VMEM((KPT, C), jnp.float32),
          pltpu.VMEM((KPT, C), jnp.float32),
          pltpu.VMEM_SHARED((BPC * F,), jnp.float32),
          pltpu.VMEM_SHARED((BPC * K,), jnp.int32),
          pltpu.SemaphoreType.DMA,
          pltpu.SemaphoreType.DMA,
          pltpu.SemaphoreType.DMA,
          pltpu.SemaphoreType.DMA,
          pltpu.SemaphoreType.DMA,
          pltpu.SemaphoreType.DMA,
          pltpu.SemaphoreType.DMA,
          pltpu.SemaphoreType.DMA,
          pltpu.SemaphoreType.DMA,
          pltpu.SemaphoreType.DMA,
          pltpu.SemaphoreType.DMA,
      ),
      compiler_params=pltpu.CompilerParams(use_tc_tiling_on_sc=False,
                                           needs_layout_passes=False),
  )


def kernel(tokens, input_2d_poses):
  poses2d = input_2d_poses.transpose(0, 2, 3, 1).reshape(B * P, F)
  tokens_flat = tokens.transpose(0, 2, 1, 3).reshape(B * J * F, C)
  out_tok, out_idx = _build()(poses2d, tokens_flat)
  out = out_tok.reshape(B, J, K, C).transpose(0, 2, 1, 3)
  return out, out_idx.reshape(B, K)


# speedup vs baseline: 1.0236x; 1.0203x over previous
import functools

import jax
import jax.numpy as jnp
from jax import lax
from jax.experimental import pallas as pl
from jax.experimental.pallas import tpu as pltpu
from jax.experimental.pallas import tpu_sc as plsc

B = 8
F = 2048
J = 17
C = 128
P = J * 2
K = 512
NC = 2
NS = 16
BPC = B // NC
QF = F // 4
GQ = QF // 16
GF = F // 16
KPT = K // 4
SCOLS = QF + 8
NBUF = 5


def _body(poses_hbm, tokens_hbm, out_tok, out_idx,
          pose_v, scores_v, sc_all, idx_v, gidx_v, gidx2, hist_v, suf_v,
          bufs, scores_sh, idx_sh, sem, gsems, ssems):
  c = lax.axis_index("c")
  s = lax.axis_index("s")
  lanes = lax.iota(jnp.int32, 16)

  bl = s // 4
  q = s % 4
  b = c * BPC + bl
  tstart = q * 4 - jnp.where(q > 0, 1, 0)
  stages = []
  for j in range(J):
    stages.append(pltpu.make_async_copy(
        poses_hbm.at[pl.ds((b * J + j) * 32 + tstart * 2, 10)],
        pose_v.at[pl.ds(j * 10, 10)], sem))
  for cp in stages:
    cp.start()
  for cp in stages:
    cp.wait()

  fb = tstart * 128

  def score_group(g, _):
    f = q * QF + g * 16 + lanes
    flc = f - fb
    flp = jnp.maximum(f - 1, 0) - fb
    rowc = lax.shift_right_logical(flc, 7) * 2
    rowp = lax.shift_right_logical(flp, 7) * 2
    colc = flc & 127
    colp = flp & 127
    acc0 = jnp.zeros((16,), jnp.float32)
    acc1 = jnp.zeros((16,), jnp.float32)
    for j in range(J):
      for o in (0, 1):
        ro = j * 10 + o
        cur = plsc.load_gather(pose_v, [rowc + ro, colc])
        prv = plsc.load_gather(pose_v, [rowp + ro, colp])
        if o == 0:
          acc0 = acc0 + jnp.abs(cur - prv)
        else:
          acc1 = acc1 + jnp.abs(cur - prv)
    scores_v[pl.ds(g * 16, 16)] = acc0 + acc1
    return 0

  lax.fori_loop(0, GQ, score_group, 0)
  pltpu.sync_copy(scores_v,
                  scores_sh.at[pl.ds(pl.multiple_of(bl * F + q * QF, QF), QF)])
  plsc.subcore_barrier()

  @pl.when(s < BPC)
  def _phase_b():
    pltpu.sync_copy(scores_sh.at[pl.ds(pl.multiple_of(s * F, F), F)], sc_all)

    ones16 = jnp.full((16,), 1, jnp.int32)
    zeros16 = jnp.zeros((16,), jnp.int32)
    pref = jnp.int32(0)
    krem = jnp.int32(K)
    rounds = ((None, 23, 0xff), (23, 15, 0xff), (15, 7, 0xff),
              (7, None, 0x7f))
    for keep_sh, dig_sh, dig_mask in rounds:
      def zbody(i, _):
        for u in range(4):
          hist_v[pl.ds((i * 4 + u) * 16, 16)] = zeros16
        return 0
      lax.fori_loop(0, 64, zbody, 0)

      def cbody(i, _):
        for u in range(4):
          g = i * 4 + u
          bits = plsc.bitcast(sc_all[pl.ds(g * 16, 16)], jnp.int32)
          if dig_sh is None:
            digit = bits & dig_mask
          else:
            digit = lax.shift_right_logical(bits, dig_sh) & dig_mask
          mask = (None if keep_sh is None
                  else lax.shift_right_logical(bits, keep_sh) == pref)
          plsc.addupdate_scatter(hist_v, [lanes * 256 + digit], ones16,
                                 mask=mask)
        return 0
      lax.fori_loop(0, GF // 4, cbody, 0)

      cum = jnp.int32(0)
      nsat = jnp.int32(0)
      for t in range(15, -1, -1):
        v = [hist_v[pl.ds(l * 256 + t * 16, 16)] for l in range(16)]
        while len(v) > 1:
          v = [v[i] + v[i + 1] for i in range(0, len(v), 2)]
        tot16 = v[0]
        suf16 = lax.rev(plsc.cumsum(lax.rev(tot16, (0,))), (0,)) + cum
        suf_v[pl.ds(t * 16, 16)] = suf16
        nsat = nsat + jnp.sum(jnp.where(suf16 >= krem, 1, 0))
        cum = jnp.max(suf16)
      dstar = nsat - 1
      nxt = jnp.minimum(dstar + 1, 255)
      s_next = jnp.where(
          dstar >= 255, 0,
          jnp.max(plsc.load_gather(suf_v, [jnp.full((16,), 0, jnp.int32)
                                           + nxt])))
      krem = krem - s_next
      if keep_sh is None:
        pref = dstar
      elif dig_sh is not None:
        pref = (pref << 8) | dstar
      else:
        pref = (pref << 7) | dstar
    thr = pref
    thr_v = jnp.full((16,), thr, jnp.int32)

    def compact(g, carry):
      off, equota = carry
      sv = plsc.bitcast(sc_all[pl.ds(g * 16, 16)], jnp.int32)
      fidx = g * 16 + lanes
      m_gt = sv > thr_v
      m_eq = sv == thr_v
      eq_rank = plsc.cumsum(jnp.where(m_eq, 1, 0))
      m_eq_sel = m_eq & (eq_rank <= equota)
      m = m_gt | m_eq_sel
      mi = jnp.where(m, 1, 0)
      pos = off + plsc.cumsum(mi) - 1
      plsc.store_scatter(idx_v, [pos], fidx, mask=m)
      return (off + jnp.sum(mi),
              equota - jnp.sum(jnp.where(m_eq_sel, 1, 0)))

    lax.fori_loop(0, GF, compact, (jnp.int32(0), krem))
    gb = c * BPC + s
    pltpu.sync_copy(idx_v, out_idx.at[pl.ds(pl.multiple_of(gb * K, K), K)])
    pltpu.sync_copy(idx_v, idx_sh.at[pl.ds(pl.multiple_of(s * K, K), K)])

  plsc.subcore_barrier()

  pltpu.sync_copy(
      idx_sh.at[pl.ds(pl.multiple_of(bl * K + q * KPT, KPT), KPT)], gidx_v)
  for j in range(J):
    rbase = (b * J + j) * F
    for t in range(KPT // 16):
      gidx2[j, pl.ds(t * 16, 16)] = gidx_v[pl.ds(t * 16, 16)] + rbase

  out0 = b * (J * K) + q * KPT
  gathers = []
  stores = []
  for j in range(J):
    gathers.append(pltpu.make_async_copy(
        tokens_hbm.at[gidx2.at[j]], bufs[j % NBUF], gsems[j % NBUF]))
    stores.append(pltpu.make_async_copy(
        bufs[j % NBUF],
        out_tok.at[pl.ds(pl.multiple_of(out0 + j * K, KPT), KPT)],
        ssems[j % NBUF]))
  gathers[0].start()
  for j in range(J):
    if j + 1 < J:
      if j + 1 - NBUF >= 0:
        stores[j + 1 - NBUF].wait()
      gathers[j + 1].start()
    gathers[j].wait()
    stores[j].start()
  for j in range(max(0, J - NBUF), J):
    stores[j].wait()


def _wrapped(poses_hbm, tokens_hbm, out_tok, out_idx,
             pose_v, scores_v, sc_all, idx_v, gidx_v, gidx2, hist_v, suf_v,
             rb0, rb1, rb2, rb3, rb4, scores_sh, idx_sh, sem,
             gsem0, gsem1, gsem2, gsem3, gsem4,
             ssem0, ssem1, ssem2, ssem3, ssem4):
  _body(poses_hbm, tokens_hbm, out_tok, out_idx,
        pose_v, scores_v, sc_all, idx_v, gidx_v, gidx2, hist_v, suf_v,
        (rb0, rb1, rb2, rb3, rb4), scores_sh, idx_sh, sem,
        (gsem0, gsem1, gsem2, gsem3, gsem4),
        (ssem0, ssem1, ssem2, ssem3, ssem4))


@functools.lru_cache(maxsize=1)
def _build():
  return pl.kernel(
      _wrapped,
      out_type=(jax.ShapeDtypeStruct((B * J * K, C), jnp.float32),
                jax.ShapeDtypeStruct((B * K,), jnp.int32)),
      mesh=plsc.VectorSubcoreMesh(core_axis_name="c", subcore_axis_name="s",
                                  num_cores=NC, num_subcores=NS),
      scratch_types=(
          pltpu.VMEM((J * 10, C), jnp.float32),
          pltpu.VMEM((QF,), jnp.float32),
          pltpu.VMEM((F,), jnp.float32),
          pltpu.VMEM((K,), jnp.int32),
          pltpu.VMEM((KPT,), jnp.int32),
          pltpu.VMEM((J, KPT), jnp.int32),
          pltpu.VMEM((16 * 256,), jnp.int32),
          pltpu.VMEM((256,), jnp.int32),
          pltpu.VMEM((KPT, C), jnp.float32),
          pltpu.VMEM((KPT, C), jnp.float32),
          pltpu.VMEM((KPT, C), jnp.float32),
          pltpu.VMEM((KPT, C), jnp.float32),
          pltpu.VMEM((KPT, C), jnp.float32),
          pltpu.VMEM_SHARED((BPC * F,), jnp.float32),
          pltpu.VMEM_SHARED((BPC * K,), jnp.int32),
          pltpu.SemaphoreType.DMA,
          pltpu.SemaphoreType.DMA,
          pltpu.SemaphoreType.DMA,
          pltpu.SemaphoreType.DMA,
          pltpu.SemaphoreType.DMA,
          pltpu.SemaphoreType.DMA,
          pltpu.SemaphoreType.DMA,
          pltpu.SemaphoreType.DMA,
          pltpu.SemaphoreType.DMA,
          pltpu.SemaphoreType.DMA,
          pltpu.SemaphoreType.DMA,
      ),
      compiler_params=pltpu.CompilerParams(use_tc_tiling_on_sc=False,
                                           needs_layout_passes=False),
  )


def kernel(tokens, input_2d_poses):
  poses_nat = (input_2d_poses.transpose(0, 2, 3, 1)
               .reshape(B, J, 2, 16, 128).transpose(0, 1, 3, 2, 4)
               .reshape(B * J * 32, C))
  tokens_flat = tokens.transpose(0, 2, 1, 3).reshape(B * J * F, C)
  out_tok, out_idx = _build()(poses_nat, tokens_flat)
  out = out_tok.reshape(B, J, K, C).transpose(0, 2, 1, 3)
  return out, out_idx.reshape(B, K)
